# baseline (speedup 1.0000x reference)
import functools

import jax
import jax.numpy as jnp
from jax import lax
from jax.experimental import pallas as pl
from jax.experimental.pallas import tpu as pltpu
from jax.experimental.pallas import tpu_sc as plsc

N_LABELS = 1000
BATCH = 16384
MASKED = 200
SC_ROWS = 192
NC, NS, L = 2, 16, 16
NW = NC * NS

UNIT_COLS = 1024
N_STRIPES = SC_ROWS // 8
N_CCHUNK = BATCH // UNIT_COLS
UNITS_PER_W = N_STRIPES * N_CCHUNK // NW
TAIL_COLS = BATCH // NW
NBUF = 4
PDIST = 2

TC_RBLK = 200
TC_SBLK = 16384
TC_CBLK = 8192


def _make_sc_call():
    mesh = plsc.VectorSubcoreMesh(core_axis_name="c", subcore_axis_name="s")

    @functools.partial(
        pl.kernel,
        mesh=mesh,
        out_type=jax.ShapeDtypeStruct((MASKED, BATCH), jnp.float32),
        scratch_types=[
            pltpu.VMEM((NBUF, 8, UNIT_COLS), jnp.float32),
            pltpu.SemaphoreType.DMA((NBUF,)),
            pltpu.SemaphoreType.DMA((NBUF,)),
        ],
        compiler_params=pltpu.CompilerParams(use_tc_tiling_on_sc=True),
    )
    def run(xt_hbm, out_hbm, bufs, sin, sout):
        wid = lax.axis_index("s") * NC + lax.axis_index("c")
        ubase = wid * UNITS_PER_W

        def unit_slices(u):
            uu = ubase + u
            s = uu // N_CCHUNK
            cc = uu % N_CCHUNK
            return pl.ds(s * 8, 8), pl.ds(cc * UNIT_COLS, UNIT_COLS)

        def start_in(u, b):
            rs, cs = unit_slices(u)
            pltpu.async_copy(xt_hbm.at[rs, cs], bufs.at[b], sin.at[b])

        def wait_in(u, b):
            rs, cs = unit_slices(u)
            pltpu.make_async_copy(xt_hbm.at[rs, cs], bufs.at[b], sin.at[b]).wait()

        def start_out(u, b):
            rs, cs = unit_slices(u)
            pltpu.async_copy(bufs.at[b], out_hbm.at[rs, cs], sout.at[b])

        def wait_out(u, b):
            rs, cs = unit_slices(u)
            pltpu.make_async_copy(bufs.at[b], out_hbm.at[rs, cs], sout.at[b]).wait()

        def compute(b, ncols=UNIT_COLS):
            def col_body(c16, carry):
                c = c16 * L
                for r0 in (0, 4):
                    v0 = bufs[b, r0, pl.ds(c, L)]
                    v1 = bufs[b, r0 + 1, pl.ds(c, L)]
                    v2 = bufs[b, r0 + 2, pl.ds(c, L)]
                    v3 = bufs[b, r0 + 3, pl.ds(c, L)]
                    gmax = jnp.maximum(jnp.maximum(v0, v1),
                                       jnp.maximum(v2, v3))
                    neg1 = jnp.float32(-1.0)
                    bufs[b, r0, pl.ds(c, L)] = jnp.where(v0 == gmax, v0, neg1)
                    bufs[b, r0 + 1, pl.ds(c, L)] = jnp.where(
                        v1 == gmax, v1, neg1)
                    bufs[b, r0 + 2, pl.ds(c, L)] = jnp.where(
                        v2 == gmax, v2, neg1)
                    bufs[b, r0 + 3, pl.ds(c, L)] = jnp.where(
                        v3 == gmax, v3, neg1)
                return carry

            lax.fori_loop(0, ncols // L, col_body, 0)

        for u in range(PDIST):
            start_in(u, u % NBUF)

        def outer(g, carry):
            for b in range(NBUF):
                u = g * NBUF + b
                wait_in(u, b)
                compute(b)
                start_out(u, b)
                nu = u + PDIST
                nb = (b + PDIST) % NBUF

                @pl.when(nu < UNITS_PER_W)
                def _():
                    @pl.when(u >= PDIST)
                    def _():
                        wait_out(u - PDIST, nb)
                    start_in(nu, nb)
            return carry

        lax.fori_loop(0, UNITS_PER_W // NBUF, outer, 0)
        for x in range(UNITS_PER_W - NBUF, UNITS_PER_W):
            wait_out(x, x % NBUF)

        trs = pl.ds(SC_ROWS, 8)
        tcs = pl.ds(wid * TAIL_COLS, TAIL_COLS)
        tdst = bufs.at[0, :, pl.ds(0, TAIL_COLS)]
        pltpu.async_copy(xt_hbm.at[trs, tcs], tdst, sin.at[0]).wait()
        compute(0, ncols=TAIL_COLS)
        pltpu.async_copy(tdst, out_hbm.at[trs, tcs], sout.at[0]).wait()

    return run


_sc_call = _make_sc_call()


def _tc_stream_body(x_ref, o_ref):
    o_ref[...] = x_ref[...]


def _tc_stream(xt):
    return pl.pallas_call(
        _tc_stream_body,
        grid=((N_LABELS - MASKED) // TC_RBLK, BATCH // TC_SBLK),
        in_specs=[pl.BlockSpec((TC_RBLK, TC_SBLK), lambda i, j: (i + 1, j))],
        out_specs=pl.BlockSpec((TC_RBLK, TC_SBLK), lambda i, j: (i + 1, j)),
        out_shape=jax.ShapeDtypeStruct((N_LABELS, BATCH), jnp.float32),
    )(xt)


def _tc_paste_body(scm_ref, alias_ref, o_ref):
    o_ref[...] = scm_ref[...]


def _tc_paste(scm, out1):
    return pl.pallas_call(
        _tc_paste_body,
        grid=(BATCH // TC_CBLK,),
        in_specs=[
            pl.BlockSpec((MASKED, TC_CBLK), lambda j: (0, j)),
            pl.BlockSpec(memory_space=pl.ANY),
        ],
        out_specs=pl.BlockSpec((MASKED, TC_CBLK), lambda j: (0, j)),
        out_shape=jax.ShapeDtypeStruct((N_LABELS, BATCH), jnp.float32),
        input_output_aliases={1: 0},
    )(scm, out1)


def kernel(similarities):
    xt = jnp.transpose(similarities)
    scm = _sc_call(xt)
    out1 = _tc_stream(xt)
    out_t = _tc_paste(scm, out1)
    return jnp.transpose(out_t)
